# Initial kernel scaffold; baseline (speedup 1.0000x reference)
import jax
import jax.numpy as jnp
from jax.experimental import pallas as pl


def kernel(loc, conf, landm, priors):
    raise NotImplementedError("write your pallas kernel here")


# speedup vs baseline: 1.0603x; 1.0603x over previous
import functools

import jax
import jax.numpy as jnp
from jax.experimental import pallas as pl

IMG = 1024.0
CONF_THR = 0.6
NMS_THR = 0.4
VAR0 = 0.1
VAR1 = 0.2
PRE = 2048


def _decode_body(loc_ref, conf_ref, landm_ref, priors_ref,
                 boxes_ref, scores_ref, landms_ref):
    pr = priors_ref[...]
    loc = loc_ref[...]
    landm = landm_ref[...]

    pcx, pcy = pr[0], pr[1]
    pw, ph = pr[2], pr[3]
    cx = pcx + loc[0] * VAR0 * pw
    cy = pcy + loc[1] * VAR0 * ph
    w = pw * jnp.exp(loc[2] * VAR1)
    h = ph * jnp.exp(loc[3] * VAR1)
    x1 = cx - w * 0.5
    y1 = cy - h * 0.5
    boxes_ref[...] = jnp.stack([x1, y1, x1 + w, y1 + h]) * IMG

    s = conf_ref[1, :]
    scores_ref[...] = jnp.where(s > CONF_THR, s, -1.0)[None, :]

    lx = [pcx + landm[2 * i] * VAR0 * pw for i in range(5)]
    ly = [pcy + landm[2 * i + 1] * VAR0 * ph for i in range(5)]
    pts = []
    for i in range(5):
        pts += [lx[i], ly[i]]
    landms_ref[...] = jnp.stack(pts) * IMG


def _nms_fixed(boxes, scores):
    x1, y1, x2, y2 = boxes[:, 0], boxes[:, 1], boxes[:, 2], boxes[:, 3]
    areas = (x2 - x1) * (y2 - y1)
    n = boxes.shape[0]
    idx = jnp.arange(n)

    def body(i, suppressed):
        alive = jnp.logical_not(suppressed[i])
        xx1 = jnp.maximum(x1[i], x1)
        yy1 = jnp.maximum(y1[i], y1)
        xx2 = jnp.minimum(x2[i], x2)
        yy2 = jnp.minimum(y2[i], y2)
        inter = jnp.clip(xx2 - xx1, 0.0, None) * jnp.clip(yy2 - yy1, 0.0, None)
        iou = inter / (areas[i] + areas - inter + 1e-12)
        supp = alive & (iou > NMS_THR) & (idx > i)
        return suppressed | supp

    suppressed0 = scores <= CONF_THR
    suppressed = jax.lax.fori_loop(0, n, body, suppressed0)
    return jnp.logical_not(suppressed)


def kernel(loc, conf, landm, priors):
    loc = loc[0]
    conf = conf[0]
    landm = landm[0]
    n = priors.shape[0]

    boxes_t, masked, landms_t = pl.pallas_call(
        _decode_body,
        out_shape=(
            jax.ShapeDtypeStruct((4, n), jnp.float32),
            jax.ShapeDtypeStruct((1, n), jnp.float32),
            jax.ShapeDtypeStruct((10, n), jnp.float32),
        ),
    )(loc.T, conf.T, landm.T, priors.T)
    boxes = boxes_t.T
    landmarks = landms_t.T
    masked = masked[0]

    top_scores, top_idx = jax.lax.top_k(masked, PRE)
    cand_boxes = boxes[top_idx]
    cand_landms = landmarks[top_idx]
    keep = _nms_fixed(cand_boxes, top_scores)
    out = jnp.concatenate([cand_boxes, top_scores[:, None], cand_landms], axis=1)
    return jnp.where(keep[:, None], out, 0.0)


# speedup vs baseline: 38.8035x; 36.5970x over previous
import functools

import jax
import jax.numpy as jnp
from jax.experimental import pallas as pl
from jax.experimental.pallas import tpu as pltpu

IMG = 1024.0
CONF_THR = 0.6
NMS_THR = 0.4
VAR0 = 0.1
VAR1 = 0.2
PRE = 2048


def _decode_body(loc_ref, conf_ref, landm_ref, priors_ref,
                 boxes_ref, scores_ref, landms_ref):
    pr = priors_ref[...]
    loc = loc_ref[...]
    landm = landm_ref[...]

    pcx, pcy = pr[0], pr[1]
    pw, ph = pr[2], pr[3]
    cx = pcx + loc[0] * VAR0 * pw
    cy = pcy + loc[1] * VAR0 * ph
    w = pw * jnp.exp(loc[2] * VAR1)
    h = ph * jnp.exp(loc[3] * VAR1)
    x1 = cx - w * 0.5
    y1 = cy - h * 0.5
    boxes_ref[...] = jnp.stack([x1, y1, x1 + w, y1 + h]) * IMG

    s = conf_ref[1, :]
    scores_ref[...] = jnp.where(s > CONF_THR, s, -1.0)[None, :]

    lx = [pcx + landm[2 * i] * VAR0 * pw for i in range(5)]
    ly = [pcy + landm[2 * i + 1] * VAR0 * ph for i in range(5)]
    pts = []
    for i in range(5):
        pts += [lx[i], ly[i]]
    landms_ref[...] = jnp.stack(pts) * IMG


def _nms_body(bcol_ref, brow_ref, score_ref, lrow_ref, out_ref, t_ref):
    n = PRE
    x1r = brow_ref[0:1, :]
    y1r = brow_ref[1:2, :]
    x2r = brow_ref[2:3, :]
    y2r = brow_ref[3:4, :]
    area_r = (x2r - x1r) * (y2r - y1r)

    rb = 256
    for b in range(n // rb):
        sl = pl.ds(b * rb, rb)
        x1c = bcol_ref[sl, 0:1]
        y1c = bcol_ref[sl, 1:2]
        x2c = bcol_ref[sl, 2:3]
        y2c = bcol_ref[sl, 3:4]
        area_c = (x2c - x1c) * (y2c - y1c)
        xx1 = jnp.maximum(x1c, x1r)
        yy1 = jnp.maximum(y1c, y1r)
        xx2 = jnp.minimum(x2c, x2r)
        yy2 = jnp.minimum(y2c, y2r)
        inter = (jnp.clip(xx2 - xx1, 0.0, None)
                 * jnp.clip(yy2 - yy1, 0.0, None))
        iou = inter / (area_c + area_r - inter + 1e-12)
        t_ref[sl, :] = (iou > NMS_THR).astype(jnp.float32)

    iota = jax.lax.broadcasted_iota(jnp.int32, (1, n), 1)
    alive0 = (score_ref[...] > CONF_THR).astype(jnp.float32)

    def body(i, alive):
        onehot = (iota == i).astype(jnp.float32)
        a_i = jnp.sum(alive * onehot)
        row = t_ref[pl.ds(i, 1), :]
        gt = (iota > i).astype(jnp.float32)
        return alive * (1.0 - row * gt * a_i)

    keep = jax.lax.fori_loop(0, n, body, alive0)

    out_ref[0:4, :] = brow_ref[...] * keep
    out_ref[4:5, :] = score_ref[...] * keep
    out_ref[5:15, :] = lrow_ref[...] * keep


def _nms_pallas(cand_boxes, cand_boxes_t, top_scores_row, cand_landms_t):
    out_t = pl.pallas_call(
        _nms_body,
        out_shape=jax.ShapeDtypeStruct((15, PRE), jnp.float32),
        scratch_shapes=[pltpu.VMEM((PRE, PRE), jnp.float32)],
    )(cand_boxes, cand_boxes_t, top_scores_row, cand_landms_t)
    return out_t.T


def kernel(loc, conf, landm, priors):
    loc = loc[0]
    conf = conf[0]
    landm = landm[0]
    n = priors.shape[0]

    boxes_t, masked, landms_t = pl.pallas_call(
        _decode_body,
        out_shape=(
            jax.ShapeDtypeStruct((4, n), jnp.float32),
            jax.ShapeDtypeStruct((1, n), jnp.float32),
            jax.ShapeDtypeStruct((10, n), jnp.float32),
        ),
    )(loc.T, conf.T, landm.T, priors.T)
    boxes = boxes_t.T
    masked = masked[0]

    top_scores, top_idx = jax.lax.top_k(masked, PRE)
    cand_boxes = boxes[top_idx]
    cand_landms_t = landms_t[:, top_idx]
    return _nms_pallas(cand_boxes, cand_boxes.T, top_scores[None, :],
                       cand_landms_t)


# speedup vs baseline: 42.3617x; 1.0917x over previous
import functools

import jax
import jax.numpy as jnp
from jax.experimental import pallas as pl
from jax.experimental.pallas import tpu as pltpu

IMG = 1024.0
CONF_THR = 0.6
NMS_THR = 0.4
VAR0 = 0.1
VAR1 = 0.2
PRE = 2048


def _decode_body(loc_ref, conf_ref, landm_ref, priors_ref,
                 boxes_ref, scores_ref, landms_ref):
    pr = priors_ref[...]
    loc = loc_ref[...]
    landm = landm_ref[...]

    pcx, pcy = pr[0], pr[1]
    pw, ph = pr[2], pr[3]
    cx = pcx + loc[0] * VAR0 * pw
    cy = pcy + loc[1] * VAR0 * ph
    w = pw * jnp.exp(loc[2] * VAR1)
    h = ph * jnp.exp(loc[3] * VAR1)
    x1 = cx - w * 0.5
    y1 = cy - h * 0.5
    boxes_ref[...] = jnp.stack([x1, y1, x1 + w, y1 + h]) * IMG

    s = conf_ref[1, :]
    scores_ref[...] = jnp.where(s > CONF_THR, s, -1.0)[None, :]

    lx = [pcx + landm[2 * i] * VAR0 * pw for i in range(5)]
    ly = [pcy + landm[2 * i + 1] * VAR0 * ph for i in range(5)]
    pts = []
    for i in range(5):
        pts += [lx[i], ly[i]]
    landms_ref[...] = jnp.stack(pts) * IMG


def _nms_body(bcol_ref, brow_ref, score_ref, lrow_ref, out_ref, t_ref):
    n = PRE
    x1r = brow_ref[0:1, :]
    y1r = brow_ref[1:2, :]
    x2r = brow_ref[2:3, :]
    y2r = brow_ref[3:4, :]
    area_r = (x2r - x1r) * (y2r - y1r)

    rb = 256
    for b in range(n // rb):
        sl = pl.ds(b * rb, rb)
        x1c = bcol_ref[sl, 0:1]
        y1c = bcol_ref[sl, 1:2]
        x2c = bcol_ref[sl, 2:3]
        y2c = bcol_ref[sl, 3:4]
        area_c = (x2c - x1c) * (y2c - y1c)
        xx1 = jnp.maximum(x1c, x1r)
        yy1 = jnp.maximum(y1c, y1r)
        xx2 = jnp.minimum(x2c, x2r)
        yy2 = jnp.minimum(y2c, y2r)
        inter = (jnp.clip(xx2 - xx1, 0.0, None)
                 * jnp.clip(yy2 - yy1, 0.0, None))
        iou = inter / (area_c + area_r - inter + 1e-12)
        t_ref[sl, :] = (iou > NMS_THR).astype(jnp.float32)

    nb = n // rb
    iota_b = jax.lax.broadcasted_iota(jnp.int32, (1, rb), 1)
    alive0_full = (score_ref[...] > CONF_THR).astype(jnp.float32)
    supp_acc = jnp.zeros((1, n), jnp.float32)
    keeps = []
    for b in range(nb):
        base = b * rb
        bsl = pl.ds(base, rb)
        alive_init = (alive0_full[:, base:base + rb]
                      * (supp_acc[:, base:base + rb] == 0.0))

        def body(i, alive, base=base):
            onehot = (iota_b == i).astype(jnp.float32)
            a_i = jnp.sum(alive * onehot)
            row = t_ref[pl.ds(base + i, 1), bsl]
            gt = (iota_b > i).astype(jnp.float32)
            return alive * (1.0 - row * gt * a_i)

        alive = jax.lax.fori_loop(0, rb, body, alive_init)
        keeps.append(alive)
        if b + 1 < nb:
            t_blk = t_ref[bsl, :]
            supp_acc = supp_acc + jax.lax.dot_general(
                alive, t_blk, (((1,), (0,)), ((), ())),
                preferred_element_type=jnp.float32)

    keep = jnp.concatenate(keeps, axis=1)

    out_ref[0:4, :] = brow_ref[...] * keep
    out_ref[4:5, :] = score_ref[...] * keep
    out_ref[5:15, :] = lrow_ref[...] * keep


def _nms_pallas(cand_boxes, cand_boxes_t, top_scores_row, cand_landms_t):
    out_t = pl.pallas_call(
        _nms_body,
        out_shape=jax.ShapeDtypeStruct((15, PRE), jnp.float32),
        scratch_shapes=[pltpu.VMEM((PRE, PRE), jnp.float32)],
    )(cand_boxes, cand_boxes_t, top_scores_row, cand_landms_t)
    return out_t.T


def kernel(loc, conf, landm, priors):
    loc = loc[0]
    conf = conf[0]
    landm = landm[0]
    n = priors.shape[0]

    boxes_t, masked, landms_t = pl.pallas_call(
        _decode_body,
        out_shape=(
            jax.ShapeDtypeStruct((4, n), jnp.float32),
            jax.ShapeDtypeStruct((1, n), jnp.float32),
            jax.ShapeDtypeStruct((10, n), jnp.float32),
        ),
    )(loc.T, conf.T, landm.T, priors.T)
    boxes = boxes_t.T
    masked = masked[0]

    top_scores, top_idx = jax.lax.top_k(masked, PRE)
    cand_boxes = boxes[top_idx]
    cand_landms_t = landms_t[:, top_idx]
    return _nms_pallas(cand_boxes, cand_boxes.T, top_scores[None, :],
                       cand_landms_t)


# speedup vs baseline: 65.1182x; 1.5372x over previous
import functools

import jax
import jax.numpy as jnp
from jax.experimental import pallas as pl
from jax.experimental.pallas import tpu as pltpu

IMG = 1024.0
CONF_THR = 0.6
NMS_THR = 0.4
VAR0 = 0.1
VAR1 = 0.2
PRE = 2048


def _decode_body(loc_ref, conf_ref, landm_ref, priors_ref,
                 boxes_ref, scores_ref, landms_ref):
    pr = priors_ref[...]
    loc = loc_ref[...]
    landm = landm_ref[...]

    pcx, pcy = pr[0], pr[1]
    pw, ph = pr[2], pr[3]
    cx = pcx + loc[0] * VAR0 * pw
    cy = pcy + loc[1] * VAR0 * ph
    w = pw * jnp.exp(loc[2] * VAR1)
    h = ph * jnp.exp(loc[3] * VAR1)
    x1 = cx - w * 0.5
    y1 = cy - h * 0.5
    boxes_ref[...] = jnp.stack([x1, y1, x1 + w, y1 + h]) * IMG

    s = conf_ref[1, :]
    scores_ref[...] = jnp.where(s > CONF_THR, s, -1.0)[None, :]

    lx = [pcx + landm[2 * i] * VAR0 * pw for i in range(5)]
    ly = [pcy + landm[2 * i + 1] * VAR0 * ph for i in range(5)]
    pts = []
    for i in range(5):
        pts += [lx[i], ly[i]]
    landms_ref[...] = jnp.stack(pts) * IMG


def _nms_body(bcol_ref, brow_ref, score_ref, lrow_ref, out_ref, t_ref, a_ref):
    n = PRE
    x1r = brow_ref[0:1, :]
    y1r = brow_ref[1:2, :]
    x2r = brow_ref[2:3, :]
    y2r = brow_ref[3:4, :]
    area_r = (x2r - x1r) * (y2r - y1r)

    rb = 256
    for b in range(n // rb):
        sl = pl.ds(b * rb, rb)
        x1c = bcol_ref[sl, 0:1]
        y1c = bcol_ref[sl, 1:2]
        x2c = bcol_ref[sl, 2:3]
        y2c = bcol_ref[sl, 3:4]
        area_c = (x2c - x1c) * (y2c - y1c)
        xx1 = jnp.maximum(x1c, x1r)
        yy1 = jnp.maximum(y1c, y1r)
        xx2 = jnp.minimum(x2c, x2r)
        yy2 = jnp.minimum(y2c, y2r)
        inter = (jnp.clip(xx2 - xx1, 0.0, None)
                 * jnp.clip(yy2 - yy1, 0.0, None))
        iou = inter / (area_c + area_r - inter + 1e-12)
        t_ref[sl, :] = (iou > NMS_THR).astype(jnp.float32)

    blk = 128
    nblk = n // blk
    iota_r = jax.lax.broadcasted_iota(jnp.int32, (blk, blk), 0)
    iota_c = jax.lax.broadcasted_iota(jnp.int32, (blk, blk), 1)
    triu = (iota_r < iota_c).astype(jnp.float32)
    alive0_full = (score_ref[...] > CONF_THR).astype(jnp.float32)
    supp_acc = jnp.zeros((1, n), jnp.float32)
    keeps = []
    for b in range(nblk):
        base = b * blk
        a_ref[...] = t_ref[base:base + blk, base:base + blk] * triu
        alive = (alive0_full[:, base:base + blk]
                 * (supp_acc[:, base:base + blk] == 0.0))
        for k in range(blk):
            a_k = alive[0:1, k:k + 1]
            row = a_ref[k:k + 1, :]
            alive = alive * (1.0 - row * a_k)
        keeps.append(alive)
        if b + 1 < nblk:
            t_blk = t_ref[base:base + blk, :]
            supp_acc = supp_acc + jax.lax.dot_general(
                alive, t_blk, (((1,), (0,)), ((), ())),
                preferred_element_type=jnp.float32)

    keep = jnp.concatenate(keeps, axis=1)

    out_ref[0:4, :] = brow_ref[...] * keep
    out_ref[4:5, :] = score_ref[...] * keep
    out_ref[5:15, :] = lrow_ref[...] * keep


def _nms_pallas(cand_boxes, cand_boxes_t, top_scores_row, cand_landms_t):
    out_t = pl.pallas_call(
        _nms_body,
        out_shape=jax.ShapeDtypeStruct((15, PRE), jnp.float32),
        scratch_shapes=[pltpu.VMEM((PRE, PRE), jnp.float32),
                        pltpu.VMEM((128, 128), jnp.float32)],
    )(cand_boxes, cand_boxes_t, top_scores_row, cand_landms_t)
    return out_t.T


def kernel(loc, conf, landm, priors):
    loc = loc[0]
    conf = conf[0]
    landm = landm[0]
    n = priors.shape[0]

    boxes_t, masked, landms_t = pl.pallas_call(
        _decode_body,
        out_shape=(
            jax.ShapeDtypeStruct((4, n), jnp.float32),
            jax.ShapeDtypeStruct((1, n), jnp.float32),
            jax.ShapeDtypeStruct((10, n), jnp.float32),
        ),
    )(loc.T, conf.T, landm.T, priors.T)
    boxes = boxes_t.T
    masked = masked[0]

    top_scores, top_idx = jax.lax.top_k(masked, PRE)
    cand_boxes = boxes[top_idx]
    cand_landms_t = landms_t[:, top_idx]
    return _nms_pallas(cand_boxes, cand_boxes.T, top_scores[None, :],
                       cand_landms_t)


# speedup vs baseline: 139.0554x; 2.1354x over previous
import functools

import jax
import jax.numpy as jnp
from jax.experimental import pallas as pl
from jax.experimental.pallas import tpu as pltpu

IMG = 1024.0
CONF_THR = 0.6
NMS_THR = 0.4
VAR0 = 0.1
VAR1 = 0.2
PRE = 2048


def _decode_body(loc_ref, conf_ref, landm_ref, priors_ref,
                 boxes_ref, scores_ref, landms_ref):
    pr = priors_ref[...]
    loc = loc_ref[...]
    landm = landm_ref[...]

    pcx, pcy = pr[0], pr[1]
    pw, ph = pr[2], pr[3]
    cx = pcx + loc[0] * VAR0 * pw
    cy = pcy + loc[1] * VAR0 * ph
    w = pw * jnp.exp(loc[2] * VAR1)
    h = ph * jnp.exp(loc[3] * VAR1)
    x1 = cx - w * 0.5
    y1 = cy - h * 0.5
    boxes_ref[...] = jnp.stack([x1, y1, x1 + w, y1 + h]) * IMG

    s = conf_ref[1, :]
    scores_ref[...] = jnp.where(s > CONF_THR, s, -1.0)[None, :]

    lx = [pcx + landm[2 * i] * VAR0 * pw for i in range(5)]
    ly = [pcy + landm[2 * i + 1] * VAR0 * ph for i in range(5)]
    pts = []
    for i in range(5):
        pts += [lx[i], ly[i]]
    landms_ref[...] = jnp.stack(pts) * IMG


def _nms_body(bcol_ref, brow_ref, score_ref, lrow_ref, out_ref, t_ref):
    n = PRE
    x1r = brow_ref[0:1, :]
    y1r = brow_ref[1:2, :]
    x2r = brow_ref[2:3, :]
    y2r = brow_ref[3:4, :]
    area_r = (x2r - x1r) * (y2r - y1r)

    rb = 256
    for b in range(n // rb):
        sl = pl.ds(b * rb, rb)
        x1c = bcol_ref[sl, 0:1]
        y1c = bcol_ref[sl, 1:2]
        x2c = bcol_ref[sl, 2:3]
        y2c = bcol_ref[sl, 3:4]
        area_c = (x2c - x1c) * (y2c - y1c)
        xx1 = jnp.maximum(x1c, x1r)
        yy1 = jnp.maximum(y1c, y1r)
        xx2 = jnp.minimum(x2c, x2r)
        yy2 = jnp.minimum(y2c, y2r)
        inter = (jnp.clip(xx2 - xx1, 0.0, None)
                 * jnp.clip(yy2 - yy1, 0.0, None))
        iou = inter / (area_c + area_r - inter + 1e-12)
        t_ref[sl, :] = (iou > NMS_THR).astype(jnp.float32)

    blk = 128
    nblk = n // blk
    iota_r = jax.lax.broadcasted_iota(jnp.int32, (blk, blk), 0)
    iota_c = jax.lax.broadcasted_iota(jnp.int32, (blk, blk), 1)
    triu = (iota_r < iota_c).astype(jnp.float32)
    alive0_full = (score_ref[...] > CONF_THR).astype(jnp.float32)
    supp_acc = jnp.zeros((1, n), jnp.float32)
    keeps = []

    def fp_cond(carry):
        x, y = carry
        return jnp.max(jnp.abs(x - y)) > 0.0

    for b in range(nblk):
        base = b * blk
        tu = t_ref[base:base + blk, base:base + blk] * triu
        alive0 = (alive0_full[:, base:base + blk]
                  * (supp_acc[:, base:base + blk] == 0.0))

        def fp_step(x, tu=tu, alive0=alive0):
            s = jax.lax.dot_general(x, tu, (((1,), (0,)), ((), ())),
                                    preferred_element_type=jnp.float32)
            return alive0 * (s == 0.0)

        def fp_body(carry, fp_step=fp_step):
            _, y = carry
            return (y, fp_step(y))

        x0 = alive0
        y0 = fp_step(x0)
        _, alive = jax.lax.while_loop(fp_cond, fp_body, (x0, y0))
        keeps.append(alive)
        if b + 1 < nblk:
            t_blk = t_ref[base:base + blk, :]
            supp_acc = supp_acc + jax.lax.dot_general(
                alive, t_blk, (((1,), (0,)), ((), ())),
                preferred_element_type=jnp.float32)

    keep = jnp.concatenate(keeps, axis=1)

    out_ref[0:4, :] = brow_ref[...] * keep
    out_ref[4:5, :] = score_ref[...] * keep
    out_ref[5:15, :] = lrow_ref[...] * keep


def _nms_pallas(cand_boxes, cand_boxes_t, top_scores_row, cand_landms_t):
    out_t = pl.pallas_call(
        _nms_body,
        out_shape=jax.ShapeDtypeStruct((15, PRE), jnp.float32),
        scratch_shapes=[pltpu.VMEM((PRE, PRE), jnp.float32)],
    )(cand_boxes, cand_boxes_t, top_scores_row, cand_landms_t)
    return out_t.T


def kernel(loc, conf, landm, priors):
    loc = loc[0]
    conf = conf[0]
    landm = landm[0]
    n = priors.shape[0]

    boxes_t, masked, landms_t = pl.pallas_call(
        _decode_body,
        out_shape=(
            jax.ShapeDtypeStruct((4, n), jnp.float32),
            jax.ShapeDtypeStruct((1, n), jnp.float32),
            jax.ShapeDtypeStruct((10, n), jnp.float32),
        ),
    )(loc.T, conf.T, landm.T, priors.T)
    boxes = boxes_t.T
    masked = masked[0]

    top_scores, top_idx = jax.lax.top_k(masked, PRE)
    cand_boxes = boxes[top_idx]
    cand_landms_t = landms_t[:, top_idx]
    return _nms_pallas(cand_boxes, cand_boxes.T, top_scores[None, :],
                       cand_landms_t)
